# baseline (speedup 1.0000x reference)
import functools

import jax
import jax.numpy as jnp
from jax import lax
from jax.experimental import pallas as pl
from jax.experimental.pallas import tpu as pltpu
from jax.experimental.pallas import tpu_sc as plsc

_DIM = 512
_NB = 2
_CB = 64


@functools.cache
def _make_lookup(B, D, V):
    info = plsc.get_sparse_core_info()
    NC, NS = info.num_cores, info.num_subcores
    NW = NC * NS
    DH = D // 2
    assert B % (NW // 2) == 0
    b_per_w = B // (NW // 2)
    NCH = b_per_w // _CB
    assert b_per_w % (_NB * _CB) == 0
    NP = NCH // _NB
    L = 16
    mesh = plsc.VectorSubcoreMesh(core_axis_name="c", subcore_axis_name="s")

    @functools.partial(
        pl.kernel,
        mesh=mesh,
        out_type=jax.ShapeDtypeStruct((B, D), jnp.float32),
        scratch_types=[
            pltpu.VMEM((b_per_w,), jnp.int32),
            pltpu.VMEM((V, DH), jnp.float32),
            [pltpu.VMEM((_CB, DH), jnp.float32) for _ in range(_NB)],
            [pltpu.SemaphoreType.DMA for _ in range(_NB)],
        ],
    )
    def lookup(table_hbm, idx_hbm, out_hbm, idx_v, tbl_v, rows, ss):
        wid = lax.axis_index("s") * NC + lax.axis_index("c")
        span = wid // 2
        half = wid % 2
        base = span * b_per_w
        col = half * DH
        pltpu.sync_copy(idx_hbm.at[pl.ds(base, b_per_w)], idx_v)
        pltpu.sync_copy(table_hbm.at[:, pl.ds(col, DH)], tbl_v)

        def build(c, jb):
            off = c * _CB

            def grp_body(g, carry):
                vec = idx_v[pl.ds(off + g * L, L)]
                for k in range(L):
                    r = vec[k]
                    i = g * L + k
                    for jj in range(DH // L):
                        rows[jb][i, pl.ds(jj * L, L)] = (
                            tbl_v[r, pl.ds(jj * L, L)])
                return carry

            lax.fori_loop(0, _CB // L, grp_body, 0)

        def scatter(c, jb):
            pltpu.async_copy(
                rows[jb],
                out_hbm.at[pl.ds(base + c * _CB, _CB), pl.ds(col, DH)],
                ss[jb])

        def scatter_wait(c, jb):
            pltpu.make_async_copy(
                rows[jb],
                out_hbm.at[pl.ds(base + c * _CB, _CB), pl.ds(col, DH)],
                ss[jb]).wait()

        def body(p, carry):
            for jb in range(_NB):
                c = _NB * p + jb

                @pl.when(c >= _NB)
                def _():
                    scatter_wait(c - _NB, jb)

                build(c, jb)
                scatter(c, jb)
            return carry

        lax.fori_loop(0, NP, body, 0)
        for jb in range(_NB):
            scatter_wait(NCH - _NB + jb, jb)

    return lookup


def kernel(ids, kernel):
    rows, cols = ids.shape
    B = rows * cols
    idx = ids.reshape(B).astype(jnp.int32)
    out = _make_lookup(B, _DIM, kernel.shape[0])(kernel, idx)
    return out.reshape(rows, cols, _DIM)


# speedup vs baseline: 1.6382x; 1.6382x over previous
import functools

import jax
import jax.numpy as jnp
from jax import lax
from jax.experimental import pallas as pl
from jax.experimental.pallas import tpu as pltpu
from jax.experimental.pallas import tpu_sc as plsc

_DIM = 512
_NB = 2
_CB = 64


@functools.cache
def _make_lookup(B, D, V):
    info = plsc.get_sparse_core_info()
    NC, NS = info.num_cores, info.num_subcores
    NW = NC * NS
    DH = D // 2
    assert B % (NW // 2) == 0
    b_per_w = B // (NW // 2)
    NCH = b_per_w // _CB
    assert b_per_w % (_NB * _CB) == 0
    NP = NCH // _NB
    L = 16
    mesh = plsc.VectorSubcoreMesh(core_axis_name="c", subcore_axis_name="s")

    @functools.partial(
        pl.kernel,
        mesh=mesh,
        out_type=jax.ShapeDtypeStruct((B, D), jnp.float32),
        scratch_types=[
            pltpu.VMEM((b_per_w,), jnp.int32),
            pltpu.VMEM((V, DH), jnp.float32),
            [pltpu.VMEM((_CB, DH), jnp.float32) for _ in range(_NB)],
            [pltpu.SemaphoreType.DMA for _ in range(_NB)],
        ],
    )
    def lookup(table_hbm, idx_hbm, out_hbm, idx_v, tbl_v, rows, ss):
        wid = lax.axis_index("s") * NC + lax.axis_index("c")
        span = wid // 2
        half = wid % 2
        base = span * b_per_w
        col = half * DH
        pltpu.sync_copy(idx_hbm.at[pl.ds(base, b_per_w)], idx_v)
        pltpu.sync_copy(table_hbm.at[:, pl.ds(col, DH)], tbl_v)

        def build(c, jb):
            off = c * _CB

            def grp_body(g, carry):
                vec = idx_v[pl.ds(off + g * L, L)]
                rs = [vec[k] for k in range(L)]

                @plsc.parallel_loop(0, DH // L, 1, unroll=DH // L)
                def col_body(jj):
                    for k in range(L):
                        rows[jb][g * L + k, pl.ds(jj * L, L)] = (
                            tbl_v[rs[k], pl.ds(jj * L, L)])

                return carry

            lax.fori_loop(0, _CB // L, grp_body, 0)

        def scatter(c, jb):
            pltpu.async_copy(
                rows[jb],
                out_hbm.at[pl.ds(base + c * _CB, _CB), pl.ds(col, DH)],
                ss[jb])

        def scatter_wait(c, jb):
            pltpu.make_async_copy(
                rows[jb],
                out_hbm.at[pl.ds(base + c * _CB, _CB), pl.ds(col, DH)],
                ss[jb]).wait()

        def body(p, carry):
            for jb in range(_NB):
                c = _NB * p + jb

                @pl.when(c >= _NB)
                def _():
                    scatter_wait(c - _NB, jb)

                build(c, jb)
                scatter(c, jb)
            return carry

        lax.fori_loop(0, NP, body, 0)
        for jb in range(_NB):
            scatter_wait(NCH - _NB + jb, jb)

    return lookup


def kernel(ids, kernel):
    rows, cols = ids.shape
    B = rows * cols
    idx = ids.reshape(B).astype(jnp.int32)
    out = _make_lookup(B, _DIM, kernel.shape[0])(kernel, idx)
    return out.reshape(rows, cols, _DIM)
